# initial kernel scaffold (unmeasured)
import jax
import jax.numpy as jnp
from jax import lax
from jax.experimental import pallas as pl
from jax.experimental.pallas import tpu as pltpu

N_DEV = 8


def kernel(x, w_mat):
    M, K = x.shape
    _, N = w_mat.shape
    CH = M // N_DEV
    H = N_DEV - 1

    def body(x_ref, w_ref, out_ref, comm_ref, send_sems, recv_sems, credit_sems):
        my = lax.axis_index("i")
        left = (my - 1) % N_DEV
        right = (my + 1) % N_DEV

        barrier = pltpu.get_barrier_semaphore()
        for nbr in (left, right):
            pl.semaphore_signal(
                barrier, inc=1, device_id=(nbr,),
                device_id_type=pl.DeviceIdType.MESH,
            )
        pl.semaphore_wait(barrier, 2)

        def chunk(i):
            return pl.ds(pl.multiple_of(i * CH, CH), CH)

        for j in range(N_DEV):
            c = (my - j) % N_DEV
            out_ref[chunk(c), :] = jnp.dot(
                x_ref[chunk(c), :], w_ref[...],
                preferred_element_type=jnp.float32,
            ).astype(out_ref.dtype)

        for h in range(2 * H):
            slot = h % 2
            if h < H:
                c_send = (my - h) % N_DEV
                c_recv = (my - h - 1) % N_DEV
            else:
                t = h - H
                c_send = (my + 1 - t) % N_DEV
                c_recv = (my - t) % N_DEV

            if h >= 2:
                pl.semaphore_wait(credit_sems.at[slot], 1)

            rdma = pltpu.make_async_remote_copy(
                src_ref=out_ref.at[chunk(c_send), :],
                dst_ref=comm_ref.at[slot],
                send_sem=send_sems.at[slot],
                recv_sem=recv_sems.at[slot],
                device_id=(right,),
                device_id_type=pl.DeviceIdType.MESH,
            )
            rdma.start()
            rdma.wait()

            if h < H:
                out_ref[chunk(c_recv), :] = (
                    out_ref[chunk(c_recv), :].astype(jnp.float32)
                    + comm_ref[slot].astype(jnp.float32)
                ).astype(out_ref.dtype)
            else:
                out_ref[chunk(c_recv), :] = comm_ref[slot]

            pl.semaphore_signal(
                credit_sems.at[slot], inc=1, device_id=(left,),
                device_id_type=pl.DeviceIdType.MESH,
            )

        pl.semaphore_wait(credit_sems.at[0], 1)
        pl.semaphore_wait(credit_sems.at[1], 1)

        amax = jnp.max(jnp.abs(out_ref[...])).astype(jnp.float32)
        scale = amax / 127.0
        inv = 127.0 / amax
        for c in range(N_DEV):
            v = out_ref[chunk(c), :].astype(jnp.float32)
            q = jnp.clip(jnp.round(v * inv), -127.0, 127.0)
            out_ref[chunk(c), :] = (q * scale).astype(out_ref.dtype)

    return pl.pallas_call(
        body,
        out_shape=jax.ShapeDtypeStruct((M, N), jnp.bfloat16),
        in_specs=[
            pl.BlockSpec(memory_space=pltpu.VMEM),
            pl.BlockSpec(memory_space=pltpu.VMEM),
        ],
        out_specs=pl.BlockSpec(memory_space=pltpu.VMEM),
        scratch_shapes=[
            pltpu.VMEM((2, CH, N), jnp.bfloat16),
            pltpu.SemaphoreType.DMA((2,)),
            pltpu.SemaphoreType.DMA((2,)),
            pltpu.SemaphoreType.REGULAR((2,)),
        ],
        compiler_params=pltpu.CompilerParams(collective_id=0),
    )(x, w_mat)


# baseline (device time: 399485 ns/iter reference)
import jax
import jax.numpy as jnp
from jax import lax
from jax.experimental import pallas as pl
from jax.experimental.pallas import tpu as pltpu

N_DEV = 8


def kernel(x, w_mat):
    M, K = x.shape
    _, N = w_mat.shape
    CH = M // N_DEV
    H = N_DEV - 1

    def body(x_ref, w_ref, out_ref, comm_ref, send_sems, recv_sems, credit_sems):
        my = lax.axis_index("i")
        left = (my - 1) % N_DEV
        right = (my + 1) % N_DEV

        barrier = pltpu.get_barrier_semaphore()
        for nbr in (left, right):
            pl.semaphore_signal(
                barrier, inc=1, device_id=(nbr,),
                device_id_type=pl.DeviceIdType.MESH,
            )
        pl.semaphore_wait(barrier, 2)

        def chunk(i):
            return pl.ds(pl.multiple_of(i * CH, CH), CH)

        for j in range(N_DEV):
            c = (my - j) % N_DEV
            out_ref[chunk(c), :] = jnp.dot(
                x_ref[chunk(c), :], w_ref[...],
                preferred_element_type=jnp.float32,
            ).astype(out_ref.dtype)

        for h in range(2 * H):
            slot = h % 2
            if h < H:
                c_send = (my - h) % N_DEV
                c_recv = (my - h - 1) % N_DEV
            else:
                t = h - H
                c_send = (my + 1 - t) % N_DEV
                c_recv = (my - t) % N_DEV

            if h >= 2:
                pl.semaphore_wait(credit_sems.at[slot], 1)

            rdma = pltpu.make_async_remote_copy(
                src_ref=out_ref.at[chunk(c_send), :],
                dst_ref=comm_ref.at[slot],
                send_sem=send_sems.at[slot],
                recv_sem=recv_sems.at[slot],
                device_id=(right,),
                device_id_type=pl.DeviceIdType.MESH,
            )
            rdma.start()
            rdma.wait()

            if h < H:
                out_ref[chunk(c_recv), :] = (
                    out_ref[chunk(c_recv), :].astype(jnp.float32)
                    + comm_ref[slot].astype(jnp.float32)
                ).astype(out_ref.dtype)
            else:
                out_ref[chunk(c_recv), :] = comm_ref[slot]

            pl.semaphore_signal(
                credit_sems.at[slot], inc=1, device_id=(left,),
                device_id_type=pl.DeviceIdType.MESH,
            )

        pl.semaphore_wait(credit_sems.at[0], 1)
        pl.semaphore_wait(credit_sems.at[1], 1)

        amax = jnp.float32(0.0)
        for c in range(N_DEV):
            amax = jnp.maximum(
                amax, jnp.max(jnp.abs(out_ref[chunk(c), :]).astype(jnp.float32))
            )
        scale = amax / 127.0
        inv = 127.0 / amax
        for c in range(N_DEV):
            v = out_ref[chunk(c), :].astype(jnp.float32)
            q = jnp.clip(jnp.round(v * inv), -127.0, 127.0)
            out_ref[chunk(c), :] = (q * scale).astype(out_ref.dtype)

    return pl.pallas_call(
        body,
        out_shape=jax.ShapeDtypeStruct((M, N), jnp.bfloat16),
        in_specs=[
            pl.BlockSpec(memory_space=pltpu.VMEM),
            pl.BlockSpec(memory_space=pltpu.VMEM),
        ],
        out_specs=pl.BlockSpec(memory_space=pltpu.VMEM),
        scratch_shapes=[
            pltpu.VMEM((2, CH, N), jnp.bfloat16),
            pltpu.SemaphoreType.DMA((2,)),
            pltpu.SemaphoreType.DMA((2,)),
            pltpu.SemaphoreType.REGULAR((2,)),
        ],
        compiler_params=pltpu.CompilerParams(collective_id=0),
    )(x, w_mat)


# device time: 250175 ns/iter; 1.5968x vs baseline; 1.5968x over previous
import jax
import jax.numpy as jnp
from jax import lax
from jax.experimental import pallas as pl
from jax.experimental.pallas import tpu as pltpu

N_DEV = 8


def kernel(x, w_mat):
    M, K = x.shape
    _, N = w_mat.shape
    CH = M // N_DEV
    NL = N // 2
    H = N_DEV - 1

    def body(x_ref, w_ref, out_ref, comm_r, comm_l,
             send_r, recv_r, send_l, recv_l, credit_r, credit_l):
        my = lax.axis_index("i")
        left = (my - 1) % N_DEV
        right = (my + 1) % N_DEV

        barrier = pltpu.get_barrier_semaphore()
        for nbr in (left, right):
            pl.semaphore_signal(
                barrier, inc=1, device_id=(nbr,),
                device_id_type=pl.DeviceIdType.MESH,
            )
        pl.semaphore_wait(barrier, 2)

        def chunk(i):
            return pl.ds(pl.multiple_of(i * CH, CH), CH)

        cols_r = pl.ds(0, NL)
        cols_l = pl.ds(NL, NL)

        for j in range(N_DEV):
            c = (my - j) % N_DEV
            out_ref[chunk(c), :] = jnp.dot(
                x_ref[chunk(c), :], w_ref[...],
                preferred_element_type=jnp.float32,
            ).astype(out_ref.dtype)

        for h in range(2 * H):
            slot = h % 2
            if h < H:
                cs_r = (my - h) % N_DEV
                cr_r = (my - h - 1) % N_DEV
                cs_l = (my + h) % N_DEV
                cr_l = (my + h + 1) % N_DEV
            else:
                t = h - H
                cs_r = (my + 1 - t) % N_DEV
                cr_r = (my - t) % N_DEV
                cs_l = (my - 1 + t) % N_DEV
                cr_l = (my + t) % N_DEV

            if h >= 2:
                pl.semaphore_wait(credit_r.at[slot], 1)
                pl.semaphore_wait(credit_l.at[slot], 1)

            rdma_right = pltpu.make_async_remote_copy(
                src_ref=out_ref.at[chunk(cs_r), cols_r],
                dst_ref=comm_r.at[slot],
                send_sem=send_r.at[slot],
                recv_sem=recv_r.at[slot],
                device_id=(right,),
                device_id_type=pl.DeviceIdType.MESH,
            )
            rdma_left = pltpu.make_async_remote_copy(
                src_ref=out_ref.at[chunk(cs_l), cols_l],
                dst_ref=comm_l.at[slot],
                send_sem=send_l.at[slot],
                recv_sem=recv_l.at[slot],
                device_id=(left,),
                device_id_type=pl.DeviceIdType.MESH,
            )
            rdma_right.start()
            rdma_left.start()
            rdma_right.wait()
            rdma_left.wait()

            if h < H:
                out_ref[chunk(cr_r), cols_r] = (
                    out_ref[chunk(cr_r), cols_r].astype(jnp.float32)
                    + comm_r[slot].astype(jnp.float32)
                ).astype(out_ref.dtype)
                out_ref[chunk(cr_l), cols_l] = (
                    out_ref[chunk(cr_l), cols_l].astype(jnp.float32)
                    + comm_l[slot].astype(jnp.float32)
                ).astype(out_ref.dtype)
            else:
                out_ref[chunk(cr_r), cols_r] = comm_r[slot]
                out_ref[chunk(cr_l), cols_l] = comm_l[slot]

            pl.semaphore_signal(
                credit_r.at[slot], inc=1, device_id=(left,),
                device_id_type=pl.DeviceIdType.MESH,
            )
            pl.semaphore_signal(
                credit_l.at[slot], inc=1, device_id=(right,),
                device_id_type=pl.DeviceIdType.MESH,
            )

        for s in (0, 1):
            pl.semaphore_wait(credit_r.at[s], 1)
            pl.semaphore_wait(credit_l.at[s], 1)

        amax = jnp.float32(0.0)
        for c in range(N_DEV):
            amax = jnp.maximum(
                amax, jnp.max(jnp.abs(out_ref[chunk(c), :]).astype(jnp.float32))
            )
        scale = amax / 127.0
        inv = 127.0 / amax
        for c in range(N_DEV):
            v = out_ref[chunk(c), :].astype(jnp.float32)
            q = jnp.clip(jnp.round(v * inv), -127.0, 127.0)
            out_ref[chunk(c), :] = (q * scale).astype(out_ref.dtype)

    return pl.pallas_call(
        body,
        out_shape=jax.ShapeDtypeStruct((M, N), jnp.bfloat16),
        in_specs=[
            pl.BlockSpec(memory_space=pltpu.VMEM),
            pl.BlockSpec(memory_space=pltpu.VMEM),
        ],
        out_specs=pl.BlockSpec(memory_space=pltpu.VMEM),
        scratch_shapes=[
            pltpu.VMEM((2, CH, NL), jnp.bfloat16),
            pltpu.VMEM((2, CH, NL), jnp.bfloat16),
            pltpu.SemaphoreType.DMA((2,)),
            pltpu.SemaphoreType.DMA((2,)),
            pltpu.SemaphoreType.DMA((2,)),
            pltpu.SemaphoreType.DMA((2,)),
            pltpu.SemaphoreType.REGULAR((2,)),
            pltpu.SemaphoreType.REGULAR((2,)),
        ],
        compiler_params=pltpu.CompilerParams(collective_id=0),
    )(x, w_mat)


# device time: 194904 ns/iter; 2.0497x vs baseline; 1.2836x over previous
import jax
import jax.numpy as jnp
from jax import lax
from jax.experimental import pallas as pl
from jax.experimental.pallas import tpu as pltpu

N_DEV = 8


def kernel(x, w_mat):
    M, K = x.shape
    _, N = w_mat.shape
    CH = M // N_DEV
    NL = N // 2
    H = N_DEV - 1

    def body(x_ref, w_ref, out_ref, comm_r, comm_l, qcomm_r, qcomm_l,
             qcarry_r, qcarry_l, amax_buf,
             send_r, recv_r, send_l, recv_l, amax_send, amax_recv,
             credit_r, credit_l):
        my = lax.axis_index("i")
        left = (my - 1) % N_DEV
        right = (my + 1) % N_DEV

        barrier = pltpu.get_barrier_semaphore()
        for nbr in (left, right):
            pl.semaphore_signal(
                barrier, inc=1, device_id=(nbr,),
                device_id_type=pl.DeviceIdType.MESH,
            )
        pl.semaphore_wait(barrier, 2)

        def chunk(i):
            return pl.ds(pl.multiple_of(i * CH, CH), CH)

        cols_r = pl.ds(0, NL)
        cols_l = pl.ds(NL, NL)

        def gemm_chunk(c):
            out_ref[chunk(c), :] = jnp.dot(
                x_ref[chunk(c), :], w_ref[...],
                preferred_element_type=jnp.float32,
            ).astype(out_ref.dtype)

        gemm_chunk(my)

        for h in range(H):
            slot = h % 2
            cs_r = (my - h) % N_DEV
            cr_r = (my - h - 1) % N_DEV
            cs_l = (my + h) % N_DEV
            cr_l = (my + h + 1) % N_DEV

            if h >= 2:
                pl.semaphore_wait(credit_r.at[slot], 1)
                pl.semaphore_wait(credit_l.at[slot], 1)

            rdma_right = pltpu.make_async_remote_copy(
                src_ref=out_ref.at[chunk(cs_r), cols_r],
                dst_ref=comm_r.at[slot],
                send_sem=send_r.at[slot],
                recv_sem=recv_r.at[slot],
                device_id=(right,),
                device_id_type=pl.DeviceIdType.MESH,
            )
            rdma_left = pltpu.make_async_remote_copy(
                src_ref=out_ref.at[chunk(cs_l), cols_l],
                dst_ref=comm_l.at[slot],
                send_sem=send_l.at[slot],
                recv_sem=recv_l.at[slot],
                device_id=(left,),
                device_id_type=pl.DeviceIdType.MESH,
            )
            rdma_right.start()
            rdma_left.start()

            if h < 3:
                gemm_chunk((my - h - 1) % N_DEV)
                gemm_chunk((my + h + 1) % N_DEV)
            elif h == 3:
                gemm_chunk((my + 4) % N_DEV)

            rdma_right.wait()
            rdma_left.wait()

            out_ref[chunk(cr_r), cols_r] = (
                out_ref[chunk(cr_r), cols_r].astype(jnp.float32)
                + comm_r[slot].astype(jnp.float32)
            ).astype(out_ref.dtype)
            out_ref[chunk(cr_l), cols_l] = (
                out_ref[chunk(cr_l), cols_l].astype(jnp.float32)
                + comm_l[slot].astype(jnp.float32)
            ).astype(out_ref.dtype)

            pl.semaphore_signal(
                credit_r.at[slot], inc=1, device_id=(left,),
                device_id_type=pl.DeviceIdType.MESH,
            )
            pl.semaphore_signal(
                credit_l.at[slot], inc=1, device_id=(right,),
                device_id_type=pl.DeviceIdType.MESH,
            )

        own_r = (my + 1) % N_DEV
        own_l = (my - 1) % N_DEV

        local_amax = jnp.maximum(
            jnp.max(jnp.abs(out_ref[chunk(own_r), cols_r]).astype(jnp.float32)),
            jnp.max(jnp.abs(out_ref[chunk(own_l), cols_l]).astype(jnp.float32)),
        )
        amax_buf[pl.ds(my, 1), :] = jnp.full((1, 128), local_amax, jnp.float32)
        amax_rdmas = []
        for o in range(1, N_DEV):
            r = pltpu.make_async_remote_copy(
                src_ref=amax_buf.at[pl.ds(my, 1), :],
                dst_ref=amax_buf.at[pl.ds(my, 1), :],
                send_sem=amax_send.at[o],
                recv_sem=amax_recv.at[o],
                device_id=((my + o) % N_DEV,),
                device_id_type=pl.DeviceIdType.MESH,
            )
            r.start()
            amax_rdmas.append(r)
        for r in amax_rdmas:
            r.wait()
        amax = jnp.max(amax_buf[...])
        scale = amax / 127.0
        inv = 127.0 / amax

        def quantize(v_bf16):
            q = jnp.clip(jnp.round(v_bf16.astype(jnp.float32) * inv),
                         -127.0, 127.0)
            return q.astype(jnp.int8)

        def dequant(q_i8):
            return (q_i8.astype(jnp.float32) * scale).astype(out_ref.dtype)

        qcarry_r[...] = quantize(out_ref[chunk(own_r), cols_r])
        qcarry_l[...] = quantize(out_ref[chunk(own_l), cols_l])
        out_ref[chunk(own_r), cols_r] = dequant(qcarry_r[...])
        out_ref[chunk(own_l), cols_l] = dequant(qcarry_l[...])

        for t in range(H):
            h = H + t
            slot = h % 2
            cr_r = (my - t) % N_DEV
            cr_l = (my + t) % N_DEV

            pl.semaphore_wait(credit_r.at[slot], 1)
            pl.semaphore_wait(credit_l.at[slot], 1)

            rdma_right = pltpu.make_async_remote_copy(
                src_ref=qcarry_r,
                dst_ref=qcomm_r.at[slot],
                send_sem=send_r.at[slot],
                recv_sem=recv_r.at[slot],
                device_id=(right,),
                device_id_type=pl.DeviceIdType.MESH,
            )
            rdma_left = pltpu.make_async_remote_copy(
                src_ref=qcarry_l,
                dst_ref=qcomm_l.at[slot],
                send_sem=send_l.at[slot],
                recv_sem=recv_l.at[slot],
                device_id=(left,),
                device_id_type=pl.DeviceIdType.MESH,
            )
            rdma_right.start()
            rdma_left.start()
            rdma_right.wait()
            rdma_left.wait()

            if t < H - 1:
                qcarry_r[...] = qcomm_r[slot]
                qcarry_l[...] = qcomm_l[slot]
            out_ref[chunk(cr_r), cols_r] = dequant(qcomm_r[slot])
            out_ref[chunk(cr_l), cols_l] = dequant(qcomm_l[slot])

            pl.semaphore_signal(
                credit_r.at[slot], inc=1, device_id=(left,),
                device_id_type=pl.DeviceIdType.MESH,
            )
            pl.semaphore_signal(
                credit_l.at[slot], inc=1, device_id=(right,),
                device_id_type=pl.DeviceIdType.MESH,
            )

        for s in (0, 1):
            pl.semaphore_wait(credit_r.at[s], 1)
            pl.semaphore_wait(credit_l.at[s], 1)

    return pl.pallas_call(
        body,
        out_shape=jax.ShapeDtypeStruct((M, N), jnp.bfloat16),
        in_specs=[
            pl.BlockSpec(memory_space=pltpu.VMEM),
            pl.BlockSpec(memory_space=pltpu.VMEM),
        ],
        out_specs=pl.BlockSpec(memory_space=pltpu.VMEM),
        scratch_shapes=[
            pltpu.VMEM((2, CH, NL), jnp.bfloat16),
            pltpu.VMEM((2, CH, NL), jnp.bfloat16),
            pltpu.VMEM((2, CH, NL), jnp.int8),
            pltpu.VMEM((2, CH, NL), jnp.int8),
            pltpu.VMEM((CH, NL), jnp.int8),
            pltpu.VMEM((CH, NL), jnp.int8),
            pltpu.VMEM((N_DEV, 128), jnp.float32),
            pltpu.SemaphoreType.DMA((2,)),
            pltpu.SemaphoreType.DMA((2,)),
            pltpu.SemaphoreType.DMA((2,)),
            pltpu.SemaphoreType.DMA((2,)),
            pltpu.SemaphoreType.DMA((N_DEV,)),
            pltpu.SemaphoreType.DMA((N_DEV,)),
            pltpu.SemaphoreType.REGULAR((2,)),
            pltpu.SemaphoreType.REGULAR((2,)),
        ],
        compiler_params=pltpu.CompilerParams(collective_id=0),
    )(x, w_mat)


# device time: 187969 ns/iter; 2.1253x vs baseline; 1.0369x over previous
import jax
import jax.numpy as jnp
from jax import lax
from jax.experimental import pallas as pl
from jax.experimental.pallas import tpu as pltpu

N_DEV = 8


def kernel(x, w_mat):
    M, K = x.shape
    _, N = w_mat.shape
    CH = M // N_DEV
    NL = N // 2
    NQ = NL // 2
    H = N_DEV - 1

    def body(x_ref, w_ref, out_ref, comm_r, comm_l, qcomm_r, qcomm_l,
             qcarry_r, qcarry_l, amax_buf,
             send_r, recv_r, send_l, recv_l, amax_send, amax_recv,
             credit_r, credit_l):
        my = lax.axis_index("i")
        left = (my - 1) % N_DEV
        right = (my + 1) % N_DEV

        barrier = pltpu.get_barrier_semaphore()
        for nbr in (left, right):
            pl.semaphore_signal(
                barrier, inc=1, device_id=(nbr,),
                device_id_type=pl.DeviceIdType.MESH,
            )
        pl.semaphore_wait(barrier, 2)

        def chunk(i):
            return pl.ds(pl.multiple_of(i * CH, CH), CH)

        cols_r = pl.ds(0, NL)
        cols_l = pl.ds(NL, NL)
        sub_r = (pl.ds(0, NQ), pl.ds(NQ, NQ))
        sub_l = (pl.ds(NL, NQ), pl.ds(NL + NQ, NQ))

        def gemm_chunk(c):
            out_ref[chunk(c), :] = jnp.dot(
                x_ref[chunk(c), :], w_ref[...],
                preferred_element_type=jnp.float32,
            ).astype(out_ref.dtype)

        gemm_chunk(my)

        for h in range(H):
            slot = h % 2
            cs_r = (my - h) % N_DEV
            cr_r = (my - h - 1) % N_DEV
            cs_l = (my + h) % N_DEV
            cr_l = (my + h + 1) % N_DEV

            if h >= 2:
                pl.semaphore_wait(credit_r.at[slot], 1)
                pl.semaphore_wait(credit_l.at[slot], 1)

            rdmas = []
            for q in (0, 1):
                rr = pltpu.make_async_remote_copy(
                    src_ref=out_ref.at[chunk(cs_r), sub_r[q]],
                    dst_ref=comm_r.at[slot, q],
                    send_sem=send_r.at[slot, q],
                    recv_sem=recv_r.at[slot, q],
                    device_id=(right,),
                    device_id_type=pl.DeviceIdType.MESH,
                )
                rl = pltpu.make_async_remote_copy(
                    src_ref=out_ref.at[chunk(cs_l), sub_l[q]],
                    dst_ref=comm_l.at[slot, q],
                    send_sem=send_l.at[slot, q],
                    recv_sem=recv_l.at[slot, q],
                    device_id=(left,),
                    device_id_type=pl.DeviceIdType.MESH,
                )
                rr.start()
                rl.start()
                rdmas.append((rr, rl))

            if h < 3:
                gemm_chunk((my - h - 1) % N_DEV)
                gemm_chunk((my + h + 1) % N_DEV)
            elif h == 3:
                gemm_chunk((my + 4) % N_DEV)

            for q in (0, 1):
                rr, rl = rdmas[q]
                rr.wait_recv()
                rl.wait_recv()
                out_ref[chunk(cr_r), sub_r[q]] = (
                    out_ref[chunk(cr_r), sub_r[q]].astype(jnp.float32)
                    + comm_r[slot, q].astype(jnp.float32)
                ).astype(out_ref.dtype)
                out_ref[chunk(cr_l), sub_l[q]] = (
                    out_ref[chunk(cr_l), sub_l[q]].astype(jnp.float32)
                    + comm_l[slot, q].astype(jnp.float32)
                ).astype(out_ref.dtype)
            for rr, rl in rdmas:
                rr.wait_send()
                rl.wait_send()

            pl.semaphore_signal(
                credit_r.at[slot], inc=1, device_id=(left,),
                device_id_type=pl.DeviceIdType.MESH,
            )
            pl.semaphore_signal(
                credit_l.at[slot], inc=1, device_id=(right,),
                device_id_type=pl.DeviceIdType.MESH,
            )

        own_r = (my + 1) % N_DEV
        own_l = (my - 1) % N_DEV

        local_amax = jnp.maximum(
            jnp.max(jnp.abs(out_ref[chunk(own_r), cols_r]).astype(jnp.float32)),
            jnp.max(jnp.abs(out_ref[chunk(own_l), cols_l]).astype(jnp.float32)),
        )
        amax_buf[pl.ds(my, 1), :] = jnp.full((1, 128), local_amax, jnp.float32)
        amax_rdmas = []
        for o in range(1, N_DEV):
            r = pltpu.make_async_remote_copy(
                src_ref=amax_buf.at[pl.ds(my, 1), :],
                dst_ref=amax_buf.at[pl.ds(my, 1), :],
                send_sem=amax_send.at[o],
                recv_sem=amax_recv.at[o],
                device_id=((my + o) % N_DEV,),
                device_id_type=pl.DeviceIdType.MESH,
            )
            r.start()
            amax_rdmas.append(r)
        for r in amax_rdmas:
            r.wait()
        amax = jnp.max(amax_buf[...])
        scale = amax / 127.0
        inv = 127.0 / amax

        def quantize(v_bf16):
            q = jnp.clip(jnp.round(v_bf16.astype(jnp.float32) * inv),
                         -127.0, 127.0)
            return q.astype(jnp.int8)

        def dequant(q_i8):
            return (q_i8.astype(jnp.float32) * scale).astype(out_ref.dtype)

        qcarry_r[...] = quantize(out_ref[chunk(own_r), cols_r])
        qcarry_l[...] = quantize(out_ref[chunk(own_l), cols_l])
        out_ref[chunk(own_r), cols_r] = dequant(qcarry_r[...])
        out_ref[chunk(own_l), cols_l] = dequant(qcarry_l[...])

        for t in range(H):
            h = H + t
            slot = h % 2

            pl.semaphore_wait(credit_r.at[slot], 1)
            pl.semaphore_wait(credit_l.at[slot], 1)

            rdma_right = pltpu.make_async_remote_copy(
                src_ref=qcarry_r,
                dst_ref=qcomm_r.at[slot],
                send_sem=send_r.at[slot, 0],
                recv_sem=recv_r.at[slot, 0],
                device_id=(right,),
                device_id_type=pl.DeviceIdType.MESH,
            )
            rdma_left = pltpu.make_async_remote_copy(
                src_ref=qcarry_l,
                dst_ref=qcomm_l.at[slot],
                send_sem=send_l.at[slot, 0],
                recv_sem=recv_l.at[slot, 0],
                device_id=(left,),
                device_id_type=pl.DeviceIdType.MESH,
            )
            rdma_right.start()
            rdma_left.start()

            if t > 0:
                out_ref[chunk((my - t + 1) % N_DEV), cols_r] = dequant(
                    qcarry_r[...]
                )
                out_ref[chunk((my + t - 1) % N_DEV), cols_l] = dequant(
                    qcarry_l[...]
                )

            rdma_right.wait()
            rdma_left.wait()

            qcarry_r[...] = qcomm_r[slot]
            qcarry_l[...] = qcomm_l[slot]

            pl.semaphore_signal(
                credit_r.at[slot], inc=1, device_id=(left,),
                device_id_type=pl.DeviceIdType.MESH,
            )
            pl.semaphore_signal(
                credit_l.at[slot], inc=1, device_id=(right,),
                device_id_type=pl.DeviceIdType.MESH,
            )

        out_ref[chunk((my - H + 1) % N_DEV), cols_r] = dequant(qcarry_r[...])
        out_ref[chunk((my + H - 1) % N_DEV), cols_l] = dequant(qcarry_l[...])

        for s in (0, 1):
            pl.semaphore_wait(credit_r.at[s], 1)
            pl.semaphore_wait(credit_l.at[s], 1)

    return pl.pallas_call(
        body,
        out_shape=jax.ShapeDtypeStruct((M, N), jnp.bfloat16),
        in_specs=[
            pl.BlockSpec(memory_space=pltpu.VMEM),
            pl.BlockSpec(memory_space=pltpu.VMEM),
        ],
        out_specs=pl.BlockSpec(memory_space=pltpu.VMEM),
        scratch_shapes=[
            pltpu.VMEM((2, 2, CH, NQ), jnp.bfloat16),
            pltpu.VMEM((2, 2, CH, NQ), jnp.bfloat16),
            pltpu.VMEM((2, CH, NL), jnp.int8),
            pltpu.VMEM((2, CH, NL), jnp.int8),
            pltpu.VMEM((CH, NL), jnp.int8),
            pltpu.VMEM((CH, NL), jnp.int8),
            pltpu.VMEM((N_DEV, 128), jnp.float32),
            pltpu.SemaphoreType.DMA((2, 2)),
            pltpu.SemaphoreType.DMA((2, 2)),
            pltpu.SemaphoreType.DMA((2, 2)),
            pltpu.SemaphoreType.DMA((2, 2)),
            pltpu.SemaphoreType.DMA((N_DEV,)),
            pltpu.SemaphoreType.DMA((N_DEV,)),
            pltpu.SemaphoreType.REGULAR((2,)),
            pltpu.SemaphoreType.REGULAR((2,)),
        ],
        compiler_params=pltpu.CompilerParams(collective_id=0),
    )(x, w_mat)


# device time: 178954 ns/iter; 2.2323x vs baseline; 1.0504x over previous
import jax
import jax.numpy as jnp
from jax import lax
from jax.experimental import pallas as pl
from jax.experimental.pallas import tpu as pltpu

N_DEV = 8


def kernel(x, w_mat):
    M, K = x.shape
    _, N = w_mat.shape
    CH = M // N_DEV
    NL = N // 2
    NQ = NL // 2
    H = N_DEV - 1

    def body(x_ref, w_ref, out_ref, comm_r, comm_l, qcomm_r, qcomm_l,
             qown_r, qown_l, amax_buf,
             send_r, recv_r, send_l, recv_l, amax_send, amax_recv,
             credit_r, credit_l):
        my = lax.axis_index("i")
        left = (my - 1) % N_DEV
        right = (my + 1) % N_DEV

        barrier = pltpu.get_barrier_semaphore()
        for nbr in (left, right):
            pl.semaphore_signal(
                barrier, inc=1, device_id=(nbr,),
                device_id_type=pl.DeviceIdType.MESH,
            )
        pl.semaphore_wait(barrier, 2)

        def chunk(i):
            return pl.ds(pl.multiple_of(i * CH, CH), CH)

        cols_r = pl.ds(0, NL)
        cols_l = pl.ds(NL, NL)
        subs = (
            (pl.ds(0, NQ), pl.ds(NQ, NQ)),
            (pl.ds(NL, NQ), pl.ds(NL + NQ, NQ)),
        )
        comms = (comm_r, comm_l)
        qcomms = (qcomm_r, qcomm_l)
        qowns = (qown_r, qown_l)
        ssems = (send_r, send_l)
        rsems = (recv_r, recv_l)
        creds = (credit_r, credit_l)

        def tgt_of(X):
            return right if X == 0 else left

        def upstream_of(X):
            return left if X == 0 else right

        def gemm_chunk(c):
            out_ref[chunk(c), :] = jnp.dot(
                x_ref[chunk(c), :], w_ref[...],
                preferred_element_type=jnp.float32,
            ).astype(out_ref.dtype)

        def mk_rs(h, X, g):
            slot = h % 2
            cs = (my - h) % N_DEV if X == 0 else (my + h) % N_DEV
            return pltpu.make_async_remote_copy(
                src_ref=out_ref.at[chunk(cs), subs[X][g]],
                dst_ref=comms[X].at[slot, g],
                send_sem=ssems[X].at[slot, g],
                recv_sem=rsems[X].at[slot, g],
                device_id=(tgt_of(X),),
                device_id_type=pl.DeviceIdType.MESH,
            )

        gemm_chunk(my)

        desc = {}
        for g in (0, 1):
            for X in (0, 1):
                d = mk_rs(0, X, g)
                d.start()
                desc[(X, g, 0)] = d

        for h in range(H):
            slot = h % 2
            if h < 3:
                gemm_chunk((my - h - 1) % N_DEV)
                gemm_chunk((my + h + 1) % N_DEV)
            elif h == 3:
                gemm_chunk((my + 4) % N_DEV)

            for g in (0, 1):
                for X in (0, 1):
                    cr = (my - h - 1) % N_DEV if X == 0 else (my + h + 1) % N_DEV
                    sub = subs[X][g]
                    d = desc[(X, g, slot)]
                    d.wait_recv()
                    out_ref[chunk(cr), sub] = (
                        out_ref[chunk(cr), sub].astype(jnp.float32)
                        + comms[X][slot, g].astype(jnp.float32)
                    ).astype(out_ref.dtype)
                    if h + 1 < H:
                        ns = (h + 1) % 2
                        if h >= 1:
                            desc[(X, g, ns)].wait_send()
                            pl.semaphore_wait(creds[X].at[ns, g], 1)
                        nd = mk_rs(h + 1, X, g)
                        nd.start()
                        desc[(X, g, ns)] = nd
                    pl.semaphore_signal(
                        creds[X].at[slot, g], inc=1,
                        device_id=(upstream_of(X),),
                        device_id_type=pl.DeviceIdType.MESH,
                    )

        for g in (0, 1):
            for X in (0, 1):
                for s in (0, 1):
                    desc[(X, g, s)].wait_send()

        own_r = (my + 1) % N_DEV
        own_l = (my - 1) % N_DEV

        local_amax = jnp.maximum(
            jnp.max(jnp.abs(out_ref[chunk(own_r), cols_r]).astype(jnp.float32)),
            jnp.max(jnp.abs(out_ref[chunk(own_l), cols_l]).astype(jnp.float32)),
        )
        amax_buf[pl.ds(my, 1), :] = jnp.full((1, 128), local_amax, jnp.float32)
        amax_rdmas = []
        for o in range(1, N_DEV):
            r = pltpu.make_async_remote_copy(
                src_ref=amax_buf.at[pl.ds(my, 1), :],
                dst_ref=amax_buf.at[pl.ds(my, 1), :],
                send_sem=amax_send.at[o],
                recv_sem=amax_recv.at[o],
                device_id=((my + o) % N_DEV,),
                device_id_type=pl.DeviceIdType.MESH,
            )
            r.start()
            amax_rdmas.append(r)
        for r in amax_rdmas:
            r.wait()
        amax = jnp.max(amax_buf[...])
        scale = amax / 127.0
        inv = 127.0 / amax

        def quantize(v_bf16):
            q = jnp.clip(jnp.round(v_bf16.astype(jnp.float32) * inv),
                         -127.0, 127.0)
            return q.astype(jnp.int8)

        def dequant(q_i8):
            return (q_i8.astype(jnp.float32) * scale).astype(out_ref.dtype)

        qown_r[...] = quantize(out_ref[chunk(own_r), cols_r])
        qown_l[...] = quantize(out_ref[chunk(own_l), cols_l])
        out_ref[chunk(own_r), cols_r] = dequant(qown_r[...])
        out_ref[chunk(own_l), cols_l] = dequant(qown_l[...])

        for t in range(H):
            h = H + t
            slot = h % 2
            hop_ds = []
            for X in (0, 1):
                pl.semaphore_wait(creds[X].at[slot, 0], 1)
                d = pltpu.make_async_remote_copy(
                    src_ref=qowns[X] if t == 0 else qcomms[X].at[1 - slot],
                    dst_ref=qcomms[X].at[slot],
                    send_sem=ssems[X].at[slot, 0],
                    recv_sem=rsems[X].at[slot, 0],
                    device_id=(tgt_of(X),),
                    device_id_type=pl.DeviceIdType.MESH,
                )
                d.start()
                hop_ds.append(d)
            if t > 0:
                out_ref[chunk((my - t + 1) % N_DEV), cols_r] = dequant(
                    qcomm_r[1 - slot]
                )
                out_ref[chunk((my + t - 1) % N_DEV), cols_l] = dequant(
                    qcomm_l[1 - slot]
                )
            for d in hop_ds:
                d.wait_recv()
                d.wait_send()
            if t > 0:
                for X in (0, 1):
                    pl.semaphore_signal(
                        creds[X].at[1 - slot, 0], inc=1,
                        device_id=(upstream_of(X),),
                        device_id_type=pl.DeviceIdType.MESH,
                    )

        out_ref[chunk((my - H + 1) % N_DEV), cols_r] = dequant(qcomm_r[1])
        out_ref[chunk((my + H - 1) % N_DEV), cols_l] = dequant(qcomm_l[1])

        for X in (0, 1):
            pl.semaphore_wait(creds[X].at[0, 0], 1)
            pl.semaphore_wait(creds[X].at[0, 1], 1)
            pl.semaphore_wait(creds[X].at[1, 1], 1)

    return pl.pallas_call(
        body,
        out_shape=jax.ShapeDtypeStruct((M, N), jnp.bfloat16),
        in_specs=[
            pl.BlockSpec(memory_space=pltpu.VMEM),
            pl.BlockSpec(memory_space=pltpu.VMEM),
        ],
        out_specs=pl.BlockSpec(memory_space=pltpu.VMEM),
        scratch_shapes=[
            pltpu.VMEM((2, 2, CH, NQ), jnp.bfloat16),
            pltpu.VMEM((2, 2, CH, NQ), jnp.bfloat16),
            pltpu.VMEM((2, CH, NL), jnp.int8),
            pltpu.VMEM((2, CH, NL), jnp.int8),
            pltpu.VMEM((CH, NL), jnp.int8),
            pltpu.VMEM((CH, NL), jnp.int8),
            pltpu.VMEM((N_DEV, 128), jnp.float32),
            pltpu.SemaphoreType.DMA((2, 2)),
            pltpu.SemaphoreType.DMA((2, 2)),
            pltpu.SemaphoreType.DMA((2, 2)),
            pltpu.SemaphoreType.DMA((2, 2)),
            pltpu.SemaphoreType.DMA((N_DEV,)),
            pltpu.SemaphoreType.DMA((N_DEV,)),
            pltpu.SemaphoreType.REGULAR((2, 2)),
            pltpu.SemaphoreType.REGULAR((2, 2)),
        ],
        compiler_params=pltpu.CompilerParams(collective_id=0),
    )(x, w_mat)


# device time: 178366 ns/iter; 2.2397x vs baseline; 1.0033x over previous
import jax
import jax.numpy as jnp
from jax import lax
from jax.experimental import pallas as pl
from jax.experimental.pallas import tpu as pltpu

N_DEV = 8
G_RS = 4
G_AG = 2


def kernel(x, w_mat):
    M, K = x.shape
    _, N = w_mat.shape
    CH = M // N_DEV
    NL = N // 2
    NQ = NL // G_RS
    NA = NL // G_AG
    H = N_DEV - 1

    def body(x_ref, w_ref, out_ref, comm_r, comm_l, qcomm_r, qcomm_l,
             qown_r, qown_l, amax_buf,
             send_r, recv_r, send_l, recv_l, amax_send, amax_recv,
             credit_r, credit_l):
        my = lax.axis_index("i")
        left = (my - 1) % N_DEV
        right = (my + 1) % N_DEV

        barrier = pltpu.get_barrier_semaphore()
        for nbr in (left, right):
            pl.semaphore_signal(
                barrier, inc=1, device_id=(nbr,),
                device_id_type=pl.DeviceIdType.MESH,
            )
        pl.semaphore_wait(barrier, 2)

        def chunk(i):
            return pl.ds(pl.multiple_of(i * CH, CH), CH)

        cols_r = pl.ds(0, NL)
        cols_l = pl.ds(NL, NL)
        base = (0, NL)
        comms = (comm_r, comm_l)
        qcomms = (qcomm_r, qcomm_l)
        qowns = (qown_r, qown_l)
        ssems = (send_r, send_l)
        rsems = (recv_r, recv_l)
        creds = (credit_r, credit_l)

        def tgt_of(X):
            return right if X == 0 else left

        def upstream_of(X):
            return left if X == 0 else right

        def rs_cols(X, g):
            return pl.ds(base[X] + g * NQ, NQ)

        def ag_cols(X, q):
            return pl.ds(base[X] + q * NA, NA)

        def gemm_chunk(c):
            out_ref[chunk(c), :] = jnp.dot(
                x_ref[chunk(c), :], w_ref[...],
                preferred_element_type=jnp.float32,
            ).astype(out_ref.dtype)

        def mk_rs(h, X, g):
            slot = h % 2
            cs = (my - h) % N_DEV if X == 0 else (my + h) % N_DEV
            return pltpu.make_async_remote_copy(
                src_ref=out_ref.at[chunk(cs), rs_cols(X, g)],
                dst_ref=comms[X].at[slot, g],
                send_sem=ssems[X].at[slot, g],
                recv_sem=rsems[X].at[slot, g],
                device_id=(tgt_of(X),),
                device_id_type=pl.DeviceIdType.MESH,
            )

        gemm_chunk(my)

        desc = {}
        for g in range(G_RS):
            for X in (0, 1):
                d = mk_rs(0, X, g)
                d.start()
                desc[(X, g, 0)] = d

        for h in range(H):
            slot = h % 2
            if h < 3:
                gemm_chunk((my - h - 1) % N_DEV)
                gemm_chunk((my + h + 1) % N_DEV)
            elif h == 3:
                gemm_chunk((my + 4) % N_DEV)

            for g in range(G_RS):
                for X in (0, 1):
                    cr = (my - h - 1) % N_DEV if X == 0 else (my + h + 1) % N_DEV
                    sub = rs_cols(X, g)
                    d = desc[(X, g, slot)]
                    d.wait_recv()
                    out_ref[chunk(cr), sub] = (
                        out_ref[chunk(cr), sub].astype(jnp.float32)
                        + comms[X][slot, g].astype(jnp.float32)
                    ).astype(out_ref.dtype)
                    if h + 1 < H:
                        ns = (h + 1) % 2
                        if h >= 1:
                            desc[(X, g, ns)].wait_send()
                            pl.semaphore_wait(creds[X].at[ns, g], 1)
                        nd = mk_rs(h + 1, X, g)
                        nd.start()
                        desc[(X, g, ns)] = nd
                    pl.semaphore_signal(
                        creds[X].at[slot, g], inc=1,
                        device_id=(upstream_of(X),),
                        device_id_type=pl.DeviceIdType.MESH,
                    )

        for g in range(G_RS):
            for X in (0, 1):
                for s in (0, 1):
                    desc[(X, g, s)].wait_send()

        own = ((my + 1) % N_DEV, (my - 1) % N_DEV)

        local_amax = jnp.maximum(
            jnp.max(jnp.abs(out_ref[chunk(own[0]), cols_r]).astype(jnp.float32)),
            jnp.max(jnp.abs(out_ref[chunk(own[1]), cols_l]).astype(jnp.float32)),
        )
        amax_buf[pl.ds(my, 1), :] = jnp.full((1, 128), local_amax, jnp.float32)
        amax_rdmas = []
        for o in range(1, N_DEV):
            r = pltpu.make_async_remote_copy(
                src_ref=amax_buf.at[pl.ds(my, 1), :],
                dst_ref=amax_buf.at[pl.ds(my, 1), :],
                send_sem=amax_send.at[o],
                recv_sem=amax_recv.at[o],
                device_id=((my + o) % N_DEV,),
                device_id_type=pl.DeviceIdType.MESH,
            )
            r.start()
            amax_rdmas.append(r)
        for r in amax_rdmas:
            r.wait()
        amax = jnp.max(amax_buf[...])
        scale = amax / 127.0
        inv = 127.0 / amax

        def quantize(v_bf16):
            q = jnp.clip(jnp.round(v_bf16.astype(jnp.float32) * inv),
                         -127.0, 127.0)
            return q.astype(jnp.int8)

        def dequant(q_i8):
            return (q_i8.astype(jnp.float32) * scale).astype(out_ref.dtype)

        qown_r[...] = quantize(out_ref[chunk(own[0]), cols_r])
        qown_l[...] = quantize(out_ref[chunk(own[1]), cols_l])

        for t in range(H):
            h = H + t
            slot = h % 2
            hop_ds = []
            for q in range(G_AG):
                for X in (0, 1):
                    pl.semaphore_wait(creds[X].at[slot, q], 1)
                    d = pltpu.make_async_remote_copy(
                        src_ref=(qowns[X].at[:, pl.ds(q * NA, NA)] if t == 0
                                 else qcomms[X].at[1 - slot, q]),
                        dst_ref=qcomms[X].at[slot, q],
                        send_sem=ssems[X].at[slot, q],
                        recv_sem=rsems[X].at[slot, q],
                        device_id=(tgt_of(X),),
                        device_id_type=pl.DeviceIdType.MESH,
                    )
                    d.start()
                    hop_ds.append((X, q, d))
            if t == 0:
                out_ref[chunk(own[0]), cols_r] = dequant(qown_r[...])
                out_ref[chunk(own[1]), cols_l] = dequant(qown_l[...])
            else:
                rows = ((my - t + 1) % N_DEV, (my + t - 1) % N_DEV)
                for q in range(G_AG):
                    for X in (0, 1):
                        out_ref[chunk(rows[X]), ag_cols(X, q)] = dequant(
                            qcomms[X][1 - slot, q]
                        )
            for _, _, d in hop_ds:
                d.wait_recv()
                d.wait_send()
            if t > 0:
                for X, q, _ in hop_ds:
                    pl.semaphore_signal(
                        creds[X].at[1 - slot, q], inc=1,
                        device_id=(upstream_of(X),),
                        device_id_type=pl.DeviceIdType.MESH,
                    )

        rows = ((my - H + 1) % N_DEV, (my + H - 1) % N_DEV)
        for q in range(G_AG):
            for X in (0, 1):
                out_ref[chunk(rows[X]), ag_cols(X, q)] = dequant(
                    qcomms[X][1, q]
                )

        for X in (0, 1):
            for q in range(G_AG):
                pl.semaphore_wait(creds[X].at[0, q], 1)
            for g in range(G_AG, G_RS):
                pl.semaphore_wait(creds[X].at[0, g], 1)
                pl.semaphore_wait(creds[X].at[1, g], 1)

    return pl.pallas_call(
        body,
        out_shape=jax.ShapeDtypeStruct((M, N), jnp.bfloat16),
        in_specs=[
            pl.BlockSpec(memory_space=pltpu.VMEM),
            pl.BlockSpec(memory_space=pltpu.VMEM),
        ],
        out_specs=pl.BlockSpec(memory_space=pltpu.VMEM),
        scratch_shapes=[
            pltpu.VMEM((2, G_RS, CH, NQ), jnp.bfloat16),
            pltpu.VMEM((2, G_RS, CH, NQ), jnp.bfloat16),
            pltpu.VMEM((2, G_AG, CH, NA), jnp.int8),
            pltpu.VMEM((2, G_AG, CH, NA), jnp.int8),
            pltpu.VMEM((CH, NL), jnp.int8),
            pltpu.VMEM((CH, NL), jnp.int8),
            pltpu.VMEM((N_DEV, 128), jnp.float32),
            pltpu.SemaphoreType.DMA((2, G_RS)),
            pltpu.SemaphoreType.DMA((2, G_RS)),
            pltpu.SemaphoreType.DMA((2, G_RS)),
            pltpu.SemaphoreType.DMA((2, G_RS)),
            pltpu.SemaphoreType.DMA((N_DEV,)),
            pltpu.SemaphoreType.DMA((N_DEV,)),
            pltpu.SemaphoreType.REGULAR((2, G_RS)),
            pltpu.SemaphoreType.REGULAR((2, G_RS)),
        ],
        compiler_params=pltpu.CompilerParams(collective_id=0),
    )(x, w_mat)


# device time: 168877 ns/iter; 2.3655x vs baseline; 1.0562x over previous
import jax
import jax.numpy as jnp
from jax import lax
from jax.experimental import pallas as pl
from jax.experimental.pallas import tpu as pltpu

N_DEV = 8
G_RS = 4
AG_DIMS = ((1, 2, 4), (2, 4, 1))
SGN = (1, -1)


def kernel(x, w_mat):
    M, K = x.shape
    _, N = w_mat.shape
    CH = M // N_DEV
    NL = N // 2
    NQ = NL // G_RS
    H = N_DEV - 1

    def body(x_ref, w_ref, out_ref, comm_r, comm_l, qag_r, qag_l, amax_buf,
             send_r, recv_r, send_l, recv_l,
             ag_send_r, ag_recv_r, ag_send_l, ag_recv_l,
             amax_send, amax_recv, credit_r, credit_l):
        my = lax.axis_index("i")
        left = (my - 1) % N_DEV
        right = (my + 1) % N_DEV

        barrier = pltpu.get_barrier_semaphore()
        for nbr in (left, right):
            pl.semaphore_signal(
                barrier, inc=1, device_id=(nbr,),
                device_id_type=pl.DeviceIdType.MESH,
            )
        pl.semaphore_wait(barrier, 2)

        def chunk(i):
            return pl.ds(pl.multiple_of(i * CH, CH), CH)

        cols = (pl.ds(0, NL), pl.ds(NL, NL))
        base = (0, NL)
        comms = (comm_r, comm_l)
        qags = (qag_r, qag_l)
        ag_ss = (ag_send_r, ag_send_l)
        ag_rs = (ag_recv_r, ag_recv_l)
        ssems = (send_r, send_l)
        rsems = (recv_r, recv_l)
        creds = (credit_r, credit_l)

        def tgt_of(X):
            return right if X == 0 else left

        def upstream_of(X):
            return left if X == 0 else right

        def rs_cols(X, g):
            return pl.ds(base[X] + g * NQ, NQ)

        def gemm_chunk(c):
            out_ref[chunk(c), :] = jnp.dot(
                x_ref[chunk(c), :], w_ref[...],
                preferred_element_type=jnp.float32,
            ).astype(out_ref.dtype)

        def mk_rs(h, X, g):
            slot = h % 2
            cs = (my - h) % N_DEV if X == 0 else (my + h) % N_DEV
            return pltpu.make_async_remote_copy(
                src_ref=out_ref.at[chunk(cs), rs_cols(X, g)],
                dst_ref=comms[X].at[slot, g],
                send_sem=ssems[X].at[slot, g],
                recv_sem=rsems[X].at[slot, g],
                device_id=(tgt_of(X),),
                device_id_type=pl.DeviceIdType.MESH,
            )

        gemm_chunk(my)

        desc = {}
        for g in range(G_RS):
            for X in (0, 1):
                d = mk_rs(0, X, g)
                d.start()
                desc[(X, g, 0)] = d

        for h in range(H):
            slot = h % 2
            if h < 3:
                gemm_chunk((my - h - 1) % N_DEV)
                gemm_chunk((my + h + 1) % N_DEV)
            elif h == 3:
                gemm_chunk((my + 4) % N_DEV)

            for g in range(G_RS):
                for X in (0, 1):
                    cr = (my - h - 1) % N_DEV if X == 0 else (my + h + 1) % N_DEV
                    sub = rs_cols(X, g)
                    d = desc[(X, g, slot)]
                    d.wait_recv()
                    out_ref[chunk(cr), sub] = (
                        out_ref[chunk(cr), sub].astype(jnp.float32)
                        + comms[X][slot, g].astype(jnp.float32)
                    ).astype(out_ref.dtype)
                    if h + 1 < H:
                        ns = (h + 1) % 2
                        if h >= 1:
                            desc[(X, g, ns)].wait_send()
                            pl.semaphore_wait(creds[X].at[ns, g], 1)
                        nd = mk_rs(h + 1, X, g)
                        nd.start()
                        desc[(X, g, ns)] = nd
                    pl.semaphore_signal(
                        creds[X].at[slot, g], inc=1,
                        device_id=(upstream_of(X),),
                        device_id_type=pl.DeviceIdType.MESH,
                    )

        for g in range(G_RS):
            for X in (0, 1):
                for s in (0, 1):
                    desc[(X, g, s)].wait_send()

        own = ((my + 1) % N_DEV, (my - 1) % N_DEV)

        local_amax = jnp.maximum(
            jnp.max(jnp.abs(out_ref[chunk(own[0]), cols[0]]).astype(jnp.float32)),
            jnp.max(jnp.abs(out_ref[chunk(own[1]), cols[1]]).astype(jnp.float32)),
        )
        amax_buf[pl.ds(my, 1), :] = jnp.full((1, 128), local_amax, jnp.float32)
        amax_rdmas = []
        for o in range(1, N_DEV):
            r = pltpu.make_async_remote_copy(
                src_ref=amax_buf.at[pl.ds(my, 1), :],
                dst_ref=amax_buf.at[pl.ds(my, 1), :],
                send_sem=amax_send.at[o],
                recv_sem=amax_recv.at[o],
                device_id=((my + o) % N_DEV,),
                device_id_type=pl.DeviceIdType.MESH,
            )
            r.start()
            amax_rdmas.append(r)
        for r in amax_rdmas:
            r.wait()
        amax = jnp.max(amax_buf[...])
        scale = amax / 127.0
        inv = 127.0 / amax

        def quantize(v_bf16):
            q = jnp.clip(jnp.round(v_bf16.astype(jnp.float32) * inv),
                         -127.0, 127.0)
            return q.astype(jnp.int8)

        def dequant(q_i8):
            return (q_i8.astype(jnp.float32) * scale).astype(out_ref.dtype)

        qag_r[chunk(own[0]), :] = quantize(out_ref[chunk(own[0]), cols[0]])
        qag_l[chunk(own[1]), :] = quantize(out_ref[chunk(own[1]), cols[1]])

        def spans_of(D):
            return ([0], [0, D[0]], [0, D[0], D[1], D[0] ^ D[1]])

        def held_chunk(X, s):
            return ((my ^ s) + SGN[X]) % N_DEV

        for k in range(3):
            step_ds = []
            for X in (0, 1):
                D = AG_DIMS[X]
                partner = my ^ D[k]
                for j, s in enumerate(spans_of(D)[k]):
                    c = held_chunk(X, s)
                    d = pltpu.make_async_remote_copy(
                        src_ref=qags[X].at[chunk(c), :],
                        dst_ref=qags[X].at[chunk(c), :],
                        send_sem=ag_ss[X].at[k, j],
                        recv_sem=ag_rs[X].at[k, j],
                        device_id=(partner,),
                        device_id_type=pl.DeviceIdType.MESH,
                    )
                    d.start()
                    step_ds.append((X, s, d))
            if k == 0:
                out_ref[chunk(own[0]), cols[0]] = dequant(qag_r[chunk(own[0]), :])
                out_ref[chunk(own[1]), cols[1]] = dequant(qag_l[chunk(own[1]), :])
            else:
                for X in (0, 1):
                    D = AG_DIMS[X]
                    for s in spans_of(D)[k - 1]:
                        c = held_chunk(X, D[k - 1] ^ s)
                        out_ref[chunk(c), cols[X]] = dequant(qags[X][chunk(c), :])
            if k < 2:
                for _, _, d in step_ds:
                    d.wait_recv()
                for _, _, d in step_ds:
                    d.wait_send()
            else:
                for X, s, d in step_ds:
                    d.wait_recv()
                    c = held_chunk(X, AG_DIMS[X][2] ^ s)
                    out_ref[chunk(c), cols[X]] = dequant(qags[X][chunk(c), :])
                for _, _, d in step_ds:
                    d.wait_send()

        for X in (0, 1):
            for g in range(G_RS):
                pl.semaphore_wait(creds[X].at[0, g], 1)
                pl.semaphore_wait(creds[X].at[1, g], 1)

    return pl.pallas_call(
        body,
        out_shape=jax.ShapeDtypeStruct((M, N), jnp.bfloat16),
        in_specs=[
            pl.BlockSpec(memory_space=pltpu.VMEM),
            pl.BlockSpec(memory_space=pltpu.VMEM),
        ],
        out_specs=pl.BlockSpec(memory_space=pltpu.VMEM),
        scratch_shapes=[
            pltpu.VMEM((2, G_RS, CH, NQ), jnp.bfloat16),
            pltpu.VMEM((2, G_RS, CH, NQ), jnp.bfloat16),
            pltpu.VMEM((M, NL), jnp.int8),
            pltpu.VMEM((M, NL), jnp.int8),
            pltpu.VMEM((N_DEV, 128), jnp.float32),
            pltpu.SemaphoreType.DMA((2, G_RS)),
            pltpu.SemaphoreType.DMA((2, G_RS)),
            pltpu.SemaphoreType.DMA((2, G_RS)),
            pltpu.SemaphoreType.DMA((2, G_RS)),
            pltpu.SemaphoreType.DMA((3, 4)),
            pltpu.SemaphoreType.DMA((3, 4)),
            pltpu.SemaphoreType.DMA((3, 4)),
            pltpu.SemaphoreType.DMA((3, 4)),
            pltpu.SemaphoreType.DMA((N_DEV,)),
            pltpu.SemaphoreType.DMA((N_DEV,)),
            pltpu.SemaphoreType.REGULAR((2, G_RS)),
            pltpu.SemaphoreType.REGULAR((2, G_RS)),
        ],
        compiler_params=pltpu.CompilerParams(collective_id=0),
    )(x, w_mat)


# device time: 168704 ns/iter; 2.3680x vs baseline; 1.0010x over previous
import jax
import jax.numpy as jnp
from jax import lax
from jax.experimental import pallas as pl
from jax.experimental.pallas import tpu as pltpu

N_DEV = 8
G_RS = 4
AG_DIMS = ((1, 2, 4), (2, 4, 1))
SGN = (1, -1)


def kernel(x, w_mat):
    M, K = x.shape
    _, N = w_mat.shape
    CH = M // N_DEV
    NL = N // 2
    NQ = NL // G_RS
    H = N_DEV - 1

    def body(x_ref, w_ref, out_ref, comm_r, comm_l, qag_r, qag_l, amax_buf,
             send_r, recv_r, send_l, recv_l,
             ag_send_r, ag_recv_r, ag_send_l, ag_recv_l,
             amax_send, amax_recv, credit_r, credit_l):
        my = lax.axis_index("i")
        left = (my - 1) % N_DEV
        right = (my + 1) % N_DEV

        barrier = pltpu.get_barrier_semaphore()
        for nbr in (left, right):
            pl.semaphore_signal(
                barrier, inc=1, device_id=(nbr,),
                device_id_type=pl.DeviceIdType.MESH,
            )
        pl.semaphore_wait(barrier, 2)

        def chunk(i):
            return pl.ds(pl.multiple_of(i * CH, CH), CH)

        cols = (pl.ds(0, NL), pl.ds(NL, NL))
        base = (0, NL)
        comms = (comm_r, comm_l)
        qags = (qag_r, qag_l)
        ag_ss = (ag_send_r, ag_send_l)
        ag_rs = (ag_recv_r, ag_recv_l)
        ssems = (send_r, send_l)
        rsems = (recv_r, recv_l)
        creds = (credit_r, credit_l)

        def tgt_of(X):
            return right if X == 0 else left

        def upstream_of(X):
            return left if X == 0 else right

        def rs_cols(X, g):
            return pl.ds(base[X] + g * NQ, NQ)

        def gemm_chunk(c):
            out_ref[chunk(c), :] = jnp.dot(
                x_ref[chunk(c), :], w_ref[...],
                preferred_element_type=jnp.float32,
            ).astype(out_ref.dtype)

        def mk_rs(h, X, g):
            slot = h % 2
            cs = (my - h) % N_DEV if X == 0 else (my + h) % N_DEV
            return pltpu.make_async_remote_copy(
                src_ref=out_ref.at[chunk(cs), rs_cols(X, g)],
                dst_ref=comms[X].at[slot, g],
                send_sem=ssems[X].at[slot, g],
                recv_sem=rsems[X].at[slot, g],
                device_id=(tgt_of(X),),
                device_id_type=pl.DeviceIdType.MESH,
            )

        gemm_chunk(my)

        desc = {}
        for g in range(G_RS):
            for X in (0, 1):
                d = mk_rs(0, X, g)
                d.start()
                desc[(X, g, 0)] = d

        for h in range(H):
            slot = h % 2
            if h < 3:
                gemm_chunk((my - h - 1) % N_DEV)
                gemm_chunk((my + h + 1) % N_DEV)
            elif h == 3:
                gemm_chunk((my + 4) % N_DEV)

            for g in range(G_RS):
                for X in (0, 1):
                    cr = (my - h - 1) % N_DEV if X == 0 else (my + h + 1) % N_DEV
                    sub = rs_cols(X, g)
                    d = desc[(X, g, slot)]
                    d.wait_recv()
                    out_ref[chunk(cr), sub] = (
                        out_ref[chunk(cr), sub] + comms[X][slot, g]
                    )
                    if h + 1 < H:
                        ns = (h + 1) % 2
                        if h >= 1:
                            desc[(X, g, ns)].wait_send()
                            pl.semaphore_wait(creds[X].at[ns, g], 1)
                        nd = mk_rs(h + 1, X, g)
                        nd.start()
                        desc[(X, g, ns)] = nd
                    pl.semaphore_signal(
                        creds[X].at[slot, g], inc=1,
                        device_id=(upstream_of(X),),
                        device_id_type=pl.DeviceIdType.MESH,
                    )

        for g in range(G_RS):
            for X in (0, 1):
                for s in (0, 1):
                    desc[(X, g, s)].wait_send()

        own = ((my + 1) % N_DEV, (my - 1) % N_DEV)

        local_amax = jnp.maximum(
            jnp.max(jnp.abs(out_ref[chunk(own[0]), cols[0]]).astype(jnp.float32)),
            jnp.max(jnp.abs(out_ref[chunk(own[1]), cols[1]]).astype(jnp.float32)),
        )
        amax_buf[pl.ds(my, 1), :] = jnp.full((1, 128), local_amax, jnp.float32)
        amax_rdmas = []
        for o in range(1, N_DEV):
            r = pltpu.make_async_remote_copy(
                src_ref=amax_buf.at[pl.ds(my, 1), :],
                dst_ref=amax_buf.at[pl.ds(my, 1), :],
                send_sem=amax_send.at[o],
                recv_sem=amax_recv.at[o],
                device_id=((my + o) % N_DEV,),
                device_id_type=pl.DeviceIdType.MESH,
            )
            r.start()
            amax_rdmas.append(r)
        for r in amax_rdmas:
            r.wait()
        amax = jnp.max(amax_buf[...])
        scale = amax / 127.0
        inv = 127.0 / amax

        def quantize(v_bf16):
            q = jnp.clip(jnp.round(v_bf16.astype(jnp.float32) * inv),
                         -127.0, 127.0)
            return q.astype(jnp.int8)

        scale_bf = scale.astype(jnp.bfloat16)

        def dequant(q_i8):
            return q_i8.astype(jnp.bfloat16) * scale_bf

        qag_r[chunk(own[0]), :] = quantize(out_ref[chunk(own[0]), cols[0]])
        qag_l[chunk(own[1]), :] = quantize(out_ref[chunk(own[1]), cols[1]])

        def spans_of(D):
            return ([0], [0, D[0]], [0, D[0], D[1], D[0] ^ D[1]])

        def held_chunk(X, s):
            return ((my ^ s) + SGN[X]) % N_DEV

        for k in range(3):
            step_ds = []
            for X in (0, 1):
                D = AG_DIMS[X]
                partner = my ^ D[k]
                for j, s in enumerate(spans_of(D)[k]):
                    c = held_chunk(X, s)
                    d = pltpu.make_async_remote_copy(
                        src_ref=qags[X].at[chunk(c), :],
                        dst_ref=qags[X].at[chunk(c), :],
                        send_sem=ag_ss[X].at[k, j],
                        recv_sem=ag_rs[X].at[k, j],
                        device_id=(partner,),
                        device_id_type=pl.DeviceIdType.MESH,
                    )
                    d.start()
                    step_ds.append((X, s, d))
            if k == 0:
                out_ref[chunk(own[0]), cols[0]] = dequant(qag_r[chunk(own[0]), :])
                out_ref[chunk(own[1]), cols[1]] = dequant(qag_l[chunk(own[1]), :])
            else:
                for X in (0, 1):
                    D = AG_DIMS[X]
                    for s in spans_of(D)[k - 1]:
                        c = held_chunk(X, D[k - 1] ^ s)
                        out_ref[chunk(c), cols[X]] = dequant(qags[X][chunk(c), :])
            if k < 2:
                for _, _, d in step_ds:
                    d.wait_recv()
                for _, _, d in step_ds:
                    d.wait_send()
            else:
                for X, s, d in step_ds:
                    d.wait_recv()
                    c = held_chunk(X, AG_DIMS[X][2] ^ s)
                    out_ref[chunk(c), cols[X]] = dequant(qags[X][chunk(c), :])
                for _, _, d in step_ds:
                    d.wait_send()

        for X in (0, 1):
            for g in range(G_RS):
                pl.semaphore_wait(creds[X].at[0, g], 1)
                pl.semaphore_wait(creds[X].at[1, g], 1)

    return pl.pallas_call(
        body,
        out_shape=jax.ShapeDtypeStruct((M, N), jnp.bfloat16),
        in_specs=[
            pl.BlockSpec(memory_space=pltpu.VMEM),
            pl.BlockSpec(memory_space=pltpu.VMEM),
        ],
        out_specs=pl.BlockSpec(memory_space=pltpu.VMEM),
        scratch_shapes=[
            pltpu.VMEM((2, G_RS, CH, NQ), jnp.bfloat16),
            pltpu.VMEM((2, G_RS, CH, NQ), jnp.bfloat16),
            pltpu.VMEM((M, NL), jnp.int8),
            pltpu.VMEM((M, NL), jnp.int8),
            pltpu.VMEM((N_DEV, 128), jnp.float32),
            pltpu.SemaphoreType.DMA((2, G_RS)),
            pltpu.SemaphoreType.DMA((2, G_RS)),
            pltpu.SemaphoreType.DMA((2, G_RS)),
            pltpu.SemaphoreType.DMA((2, G_RS)),
            pltpu.SemaphoreType.DMA((3, 4)),
            pltpu.SemaphoreType.DMA((3, 4)),
            pltpu.SemaphoreType.DMA((3, 4)),
            pltpu.SemaphoreType.DMA((3, 4)),
            pltpu.SemaphoreType.DMA((N_DEV,)),
            pltpu.SemaphoreType.DMA((N_DEV,)),
            pltpu.SemaphoreType.REGULAR((2, G_RS)),
            pltpu.SemaphoreType.REGULAR((2, G_RS)),
        ],
        compiler_params=pltpu.CompilerParams(collective_id=0),
    )(x, w_mat)
